# Initial kernel scaffold; baseline (speedup 1.0000x reference)
import jax
import jax.numpy as jnp
from jax.experimental import pallas as pl


def kernel(w, beta, x, y, particle_id):
    raise NotImplementedError("write your pallas kernel here")


# speedup vs baseline: 8.7570x; 8.7570x over previous
import functools

import jax
import jax.numpy as jnp
from jax import lax
from jax.experimental import pallas as pl
from jax.experimental.pallas import tpu as pltpu
from jax.experimental.pallas import tpu_sc as plsc

N = 100000
NSEG = 5000
TAB = 5120
NW = 32
PER = N // NW
NCH = PER // 16 + 2
WIN = NCH * 16
SB_W = 0.1


def _sc_partials(beta, pid):
    mesh = plsc.VectorSubcoreMesh(core_axis_name="c", subcore_axis_name="s")

    @functools.partial(
        pl.kernel,
        out_type=jax.ShapeDtypeStruct((NW, TAB), jnp.float32),
        mesh=mesh,
        scratch_types=[
            pltpu.VMEM((WIN,), jnp.float32),
            pltpu.VMEM((WIN,), jnp.int32),
            pltpu.VMEM((TAB,), jnp.float32),
        ],
        compiler_params=pltpu.CompilerParams(needs_layout_passes=False),
    )
    def k(beta_hbm, pid_hbm, out_hbm, beta_v, pid_v, acc_v):
        wid = lax.axis_index("s") * 2 + lax.axis_index("c")
        lo = wid * PER
        hi = lo + PER
        base = jnp.minimum((lo // 16) * 16, N - WIN)
        pltpu.sync_copy(beta_hbm.at[pl.ds(base, WIN)], beta_v)
        pltpu.sync_copy(pid_hbm.at[pl.ds(base, WIN)], pid_v)

        neg1 = jnp.full((16,), -1.0, jnp.float32)

        def init_body(i, c):
            acc_v[pl.ds(i * 16, 16)] = neg1
            return c

        lax.fori_loop(0, TAB // 16, init_body, 0)

        lane = lax.iota(jnp.int32, 16)

        def chunk(j, carry):
            ns, nc = carry
            b = beta_v[pl.ds(j * 16, 16)]
            p = pid_v[pl.ds(j * 16, 16)]
            gi = (base + j * 16) + lane
            valid = (gi >= lo) & (gi < hi)
            isn = valid & (p == 0)
            ns = ns + jnp.where(isn, b, 0.0)
            nc = nc + jnp.where(isn, 1.0, 0.0)
            pc = jnp.where(valid, p, (NSEG + 2) + lane)
            cur = plsc.load_gather(acc_v, (pc,))
            pending = b > cur

            def wcond(pend):
                return jnp.any(pend)

            def wbody(pend):
                plsc.store_scatter(acc_v, (pc,), b, mask=pend)
                cur2 = plsc.load_gather(acc_v, (pc,))
                return pend & (b > cur2)

            lax.while_loop(wcond, wbody, pending)
            return ns, nc

        zeros = jnp.zeros((16,), jnp.float32)
        ns, nc = lax.fori_loop(0, NCH, chunk, (zeros, zeros))
        ns_s = jnp.sum(ns)
        nc_s = jnp.sum(nc)
        tail = jnp.where(lane == 0, ns_s, jnp.where(lane == 1, nc_s, -1.0))
        acc_v[pl.ds(NSEG, 16)] = tail
        pltpu.sync_copy(acc_v, out_hbm.at[wid])

    return k(beta, pid)


def _tc_combine(partials):
    def body(p_ref, o_ref):
        t = p_ref[:]
        m = jnp.max(t, axis=0, keepdims=True)
        ids = lax.broadcasted_iota(jnp.int32, (1, TAB), 1)
        present = (m > -0.5) & (ids > 0) & (ids < NSEG)
        n_unique = jnp.sum(present.astype(jnp.float32))
        mean_term = jnp.sum(jnp.where(present, 1.0 - m, 0.0)) / n_unique
        ids2 = lax.broadcasted_iota(jnp.int32, (NW, TAB), 1)
        ns = jnp.sum(jnp.where(ids2 == NSEG, t, 0.0))
        nc = jnp.sum(jnp.where(ids2 == NSEG + 1, t, 0.0))
        nb_safe = jnp.where(nc > 0, nc, 1.0)
        loss = mean_term + SB_W * ns / nb_safe
        o_ref[0, 0] = jnp.where(nc > 0, loss, 0.0)

    return pl.pallas_call(
        body,
        out_shape=jax.ShapeDtypeStruct((1, 1), jnp.float32),
        out_specs=pl.BlockSpec(memory_space=pltpu.SMEM),
    )(partials)


def kernel(w, beta, x, y, particle_id):
    partials = _sc_partials(beta, particle_id)
    return _tc_combine(partials)[0, 0]
